# baseline (speedup 1.0000x reference)
import functools

import jax
import jax.numpy as jnp
from jax import lax
from jax.experimental import pallas as pl
from jax.experimental.pallas import tpu as pltpu
from jax.experimental.pallas import tpu_sc as plsc

EPS = 1e-8
ROUTER_TEMP = 1.0
NUM_EXPERTS = 16
TC_TILE = 256
NBUF = 16


def _tc_body(x_hbm, p_ref, out_ref, xbuf, sems):
    n = x_hbm.shape[0]
    n_tiles = n // TC_TILE

    def copy_in(tile, slot):
        return pltpu.make_async_copy(
            x_hbm.at[pl.ds(tile * TC_TILE, TC_TILE), :],
            xbuf.at[slot], sems.at[slot])

    for t in range(min(NBUF, n_tiles)):
        copy_in(t, t).start()

    for g in range(n_tiles):
        slot = g % NBUF
        copy_in(g, slot).wait()
        x = xbuf[slot]
        ss = jnp.sum(x * x, axis=1, keepdims=True)
        norm = jnp.maximum(jnp.sqrt(ss), EPS)
        xn = x / norm
        out_ref[:, g * TC_TILE:(g + 1) * TC_TILE] = lax.dot_general(
            p_ref[...], xn, (((1,), (1,)), ((), ())),
            preferred_element_type=jnp.float32) * (1.0 / ROUTER_TEMP)
        nxt = g + NBUF
        if nxt < n_tiles:
            copy_in(nxt, slot).start()


def _tc_scaled_logits(x, prototypes):
    n, d = x.shape
    e = prototypes.shape[0]
    return pl.pallas_call(
        _tc_body,
        in_specs=[
            pl.BlockSpec(memory_space=pl.ANY),
            pl.BlockSpec((e, d), lambda: (0, 0)),
        ],
        out_specs=pl.BlockSpec((e, n), lambda: (0, 0)),
        out_shape=jax.ShapeDtypeStruct((e, n), jnp.float32),
        scratch_shapes=[
            pltpu.VMEM((NBUF, TC_TILE, d), jnp.float32),
            pltpu.SemaphoreType.DMA((NBUF,)),
        ],
    )(x, prototypes)


def _sc_topk_call(logits_t, n_tokens):
    E = NUM_EXPERTS
    NC, NS = 2, 16
    NW = NC * NS
    C = n_tokens // NW
    G = C // 16

    mesh = plsc.VectorSubcoreMesh(core_axis_name="c", subcore_axis_name="s")

    @functools.partial(
        pl.kernel,
        mesh=mesh,
        out_type=[
            jax.ShapeDtypeStruct((n_tokens,), jnp.int32),
            jax.ShapeDtypeStruct((n_tokens,), jnp.int32),
            jax.ShapeDtypeStruct((n_tokens,), jnp.float32),
            jax.ShapeDtypeStruct((n_tokens,), jnp.float32),
        ],
        scratch_types=[
            pltpu.VMEM((E * C,), jnp.float32),
            pltpu.VMEM((C,), jnp.int32),
            pltpu.VMEM((C,), jnp.int32),
            pltpu.VMEM((C,), jnp.float32),
            pltpu.VMEM((C,), jnp.float32),
        ],
    )
    def sc_kernel(lg_hbm, i1_hbm, i2_hbm, w1_hbm, w2_hbm,
                  lg_v, i1_v, i2_v, w1_v, w2_v):
        wid = lax.axis_index("s") * NC + lax.axis_index("c")
        base = wid * C
        for e in range(E):
            pltpu.sync_copy(lg_hbm.at[e, pl.ds(base, C)],
                            lg_v.at[pl.ds(e * C, C)])

        def body(g, carry):
            t0 = g * 16
            vs = [lg_v[pl.ds(e * C + t0, 16)] for e in range(E)]
            best = vs[0]
            bi = jnp.zeros((16,), jnp.int32)
            for e in range(1, E):
                gt = vs[e] > best
                best = jnp.where(gt, vs[e], best)
                bi = jnp.where(gt, jnp.full((16,), e, jnp.int32), bi)
            best2 = jnp.full((16,), -jnp.inf, jnp.float32)
            bi2 = jnp.zeros((16,), jnp.int32)
            for e in range(E):
                ev = jnp.full((16,), e, jnp.int32)
                gt = (vs[e] > best2) & (bi != ev)
                best2 = jnp.where(gt, vs[e], best2)
                bi2 = jnp.where(gt, ev, bi2)
            ex = jnp.exp(best2 - best)
            w1 = 1.0 / (1.0 + ex)
            w2 = 1.0 - w1
            i1_v[pl.ds(t0, 16)] = bi
            i2_v[pl.ds(t0, 16)] = bi2
            w1_v[pl.ds(t0, 16)] = w1
            w2_v[pl.ds(t0, 16)] = w2
            return carry

        lax.fori_loop(0, G, body, 0)

        pltpu.sync_copy(i1_v, i1_hbm.at[pl.ds(base, C)])
        pltpu.sync_copy(i2_v, i2_hbm.at[pl.ds(base, C)])
        pltpu.sync_copy(w1_v, w1_hbm.at[pl.ds(base, C)])
        pltpu.sync_copy(w2_v, w2_hbm.at[pl.ds(base, C)])

    return sc_kernel(logits_t)


def kernel(input, prototypes, input_ids, attention_mask):
    b, s, d = input.shape
    n = b * s
    x = input.astype(prototypes.dtype).reshape(n, d)
    logits_t = _tc_scaled_logits(x, prototypes)
    i1, i2, w1, w2 = _sc_topk_call(logits_t, n)
    top_idx = jnp.stack([i1, i2], axis=-1).reshape(b, s, 2)
    top_w = jnp.stack([w1, w2], axis=-1).reshape(b, s, 2)
    return top_idx, top_w


# speedup vs baseline: 1.1384x; 1.1384x over previous
import functools

import jax
import jax.numpy as jnp
from jax import lax
from jax.experimental import pallas as pl
from jax.experimental.pallas import tpu as pltpu
from jax.experimental.pallas import tpu_sc as plsc

EPS = 1e-8
ROUTER_TEMP = 1.0
NUM_EXPERTS = 16
TC_TILE = 1024
SUB = 4
NBUF = 4


def _tc_body(x_hbm, p_ref, out_ref, xbuf, sems):
    n = x_hbm.shape[0]
    n_tiles = n // TC_TILE
    rows = TC_TILE // SUB

    def copies(tile, slot):
        return [pltpu.make_async_copy(
            x_hbm.at[pl.ds(tile * TC_TILE + k * rows, rows), :],
            xbuf.at[slot, pl.ds(k * rows, rows), :],
            sems.at[slot]) for k in range(SUB)]

    for t in range(min(NBUF, n_tiles)):
        for c in copies(t, t):
            c.start()

    for g in range(n_tiles):
        slot = g % NBUF
        for c in copies(g, slot):
            c.wait()
        x = xbuf[slot]
        ss = jnp.sum(x * x, axis=1, keepdims=True)
        norm = jnp.maximum(jnp.sqrt(ss), EPS)
        xn = x / norm
        out_ref[:, g * TC_TILE:(g + 1) * TC_TILE] = lax.dot_general(
            p_ref[...], xn, (((1,), (1,)), ((), ())),
            preferred_element_type=jnp.float32) * (1.0 / ROUTER_TEMP)
        nxt = g + NBUF
        if nxt < n_tiles:
            for c in copies(nxt, slot):
                c.start()


def _tc_scaled_logits(x, prototypes):
    n, d = x.shape
    e = prototypes.shape[0]
    return pl.pallas_call(
        _tc_body,
        in_specs=[
            pl.BlockSpec(memory_space=pl.ANY),
            pl.BlockSpec((e, d), lambda: (0, 0)),
        ],
        out_specs=pl.BlockSpec((e, n), lambda: (0, 0)),
        out_shape=jax.ShapeDtypeStruct((e, n), jnp.float32),
        scratch_shapes=[
            pltpu.VMEM((NBUF, TC_TILE, d), jnp.float32),
            pltpu.SemaphoreType.DMA((NBUF,)),
        ],
    )(x, prototypes)


def _sc_topk_call(logits_t, n_tokens):
    E = NUM_EXPERTS
    NC, NS = 2, 16
    NW = NC * NS
    C = n_tokens // NW
    G = C // 16

    mesh = plsc.VectorSubcoreMesh(core_axis_name="c", subcore_axis_name="s")

    @functools.partial(
        pl.kernel,
        mesh=mesh,
        out_type=[
            jax.ShapeDtypeStruct((n_tokens,), jnp.int32),
            jax.ShapeDtypeStruct((n_tokens,), jnp.int32),
            jax.ShapeDtypeStruct((n_tokens,), jnp.float32),
            jax.ShapeDtypeStruct((n_tokens,), jnp.float32),
        ],
        scratch_types=[
            pltpu.VMEM((E * C,), jnp.float32),
            pltpu.VMEM((C,), jnp.int32),
            pltpu.VMEM((C,), jnp.int32),
            pltpu.VMEM((C,), jnp.float32),
            pltpu.VMEM((C,), jnp.float32),
        ],
    )
    def sc_kernel(lg_hbm, i1_hbm, i2_hbm, w1_hbm, w2_hbm,
                  lg_v, i1_v, i2_v, w1_v, w2_v):
        wid = lax.axis_index("s") * NC + lax.axis_index("c")
        base = wid * C
        for e in range(E):
            pltpu.sync_copy(lg_hbm.at[e, pl.ds(base, C)],
                            lg_v.at[pl.ds(e * C, C)])

        def body(g, carry):
            t0 = g * 16
            vs = [lg_v[pl.ds(e * C + t0, 16)] for e in range(E)]
            best = vs[0]
            bi = jnp.zeros((16,), jnp.int32)
            for e in range(1, E):
                gt = vs[e] > best
                best = jnp.where(gt, vs[e], best)
                bi = jnp.where(gt, jnp.full((16,), e, jnp.int32), bi)
            best2 = jnp.full((16,), -jnp.inf, jnp.float32)
            bi2 = jnp.zeros((16,), jnp.int32)
            for e in range(E):
                ev = jnp.full((16,), e, jnp.int32)
                gt = (vs[e] > best2) & (bi != ev)
                best2 = jnp.where(gt, vs[e], best2)
                bi2 = jnp.where(gt, ev, bi2)
            ex = jnp.exp(best2 - best)
            w1 = 1.0 / (1.0 + ex)
            w2 = 1.0 - w1
            i1_v[pl.ds(t0, 16)] = bi
            i2_v[pl.ds(t0, 16)] = bi2
            w1_v[pl.ds(t0, 16)] = w1
            w2_v[pl.ds(t0, 16)] = w2
            return carry

        lax.fori_loop(0, G, body, 0)

        pltpu.sync_copy(i1_v, i1_hbm.at[pl.ds(base, C)])
        pltpu.sync_copy(i2_v, i2_hbm.at[pl.ds(base, C)])
        pltpu.sync_copy(w1_v, w1_hbm.at[pl.ds(base, C)])
        pltpu.sync_copy(w2_v, w2_hbm.at[pl.ds(base, C)])

    return sc_kernel(logits_t)


def kernel(input, prototypes, input_ids, attention_mask):
    b, s, d = input.shape
    n = b * s
    x = input.astype(prototypes.dtype).reshape(n, d)
    logits_t = _tc_scaled_logits(x, prototypes)
    i1, i2, w1, w2 = _sc_topk_call(logits_t, n)
    top_idx = jnp.stack([i1, i2], axis=-1).reshape(b, s, 2)
    top_w = jnp.stack([w1, w2], axis=-1).reshape(b, s, 2)
    return top_idx, top_w


# speedup vs baseline: 1.2611x; 1.1077x over previous
import functools

import jax
import jax.numpy as jnp
from jax import lax
from jax.experimental import pallas as pl
from jax.experimental.pallas import tpu as pltpu
from jax.experimental.pallas import tpu_sc as plsc

EPS = 1e-8
ROUTER_TEMP = 1.0
NUM_EXPERTS = 16
TC_TILE = 1024
SUB = 4
NBUF = 4


def _tc_body(x_hbm, p_ref, out_ref, xbuf, sems):
    n = x_hbm.shape[0]
    n_tiles = n // TC_TILE
    rows = TC_TILE // SUB

    def copies(tile, slot):
        return [pltpu.make_async_copy(
            x_hbm.at[pl.ds(tile * TC_TILE + k * rows, rows), :],
            xbuf.at[slot, pl.ds(k * rows, rows), :],
            sems.at[slot]) for k in range(SUB)]

    for t in range(min(NBUF, n_tiles)):
        for c in copies(t, t):
            c.start()

    for g in range(n_tiles):
        slot = g % NBUF
        for c in copies(g, slot):
            c.wait()
        x = xbuf[slot]
        ss = jnp.sum(x * x, axis=1, keepdims=True)
        norm = jnp.maximum(jnp.sqrt(ss), EPS)
        xn = x / norm
        out_ref[:, g * TC_TILE:(g + 1) * TC_TILE] = lax.dot_general(
            p_ref[...], xn, (((1,), (1,)), ((), ())),
            preferred_element_type=jnp.float32) * (1.0 / ROUTER_TEMP)
        nxt = g + NBUF
        if nxt < n_tiles:
            for c in copies(nxt, slot):
                c.start()


def _tc_scaled_logits(x, prototypes):
    n, d = x.shape
    e = prototypes.shape[0]
    return pl.pallas_call(
        _tc_body,
        in_specs=[
            pl.BlockSpec(memory_space=pl.ANY),
            pl.BlockSpec((e, d), lambda: (0, 0)),
        ],
        out_specs=pl.BlockSpec((e, n), lambda: (0, 0)),
        out_shape=jax.ShapeDtypeStruct((e, n), jnp.float32),
        scratch_shapes=[
            pltpu.VMEM((NBUF, TC_TILE, d), jnp.float32),
            pltpu.SemaphoreType.DMA((NBUF,)),
        ],
    )(x, prototypes)


def _sc_topk_call(logits_t, n_tokens):
    E = NUM_EXPERTS
    NC, NS = 2, 16
    NW = NC * NS
    C = n_tokens // NW
    G = C // 16

    mesh = plsc.VectorSubcoreMesh(core_axis_name="c", subcore_axis_name="s")

    @functools.partial(
        pl.kernel,
        mesh=mesh,
        out_type=[
            jax.ShapeDtypeStruct((n_tokens,), jnp.int32),
            jax.ShapeDtypeStruct((n_tokens,), jnp.int32),
            jax.ShapeDtypeStruct((n_tokens,), jnp.float32),
            jax.ShapeDtypeStruct((n_tokens,), jnp.float32),
        ],
        scratch_types=[
            pltpu.VMEM((E * C,), jnp.float32),
            pltpu.VMEM((C,), jnp.int32),
            pltpu.VMEM((C,), jnp.int32),
            pltpu.VMEM((C,), jnp.float32),
            pltpu.VMEM((C,), jnp.float32),
            pltpu.SemaphoreType.DMA,
            pltpu.SemaphoreType.DMA,
        ],
    )
    def sc_kernel(lg_hbm, i1_hbm, i2_hbm, w1_hbm, w2_hbm,
                  lg_v, i1_v, i2_v, w1_v, w2_v, in_sem, out_sem):
        wid = lax.axis_index("s") * NC + lax.axis_index("c")
        base = wid * C
        in_copies = [pltpu.make_async_copy(lg_hbm.at[e, pl.ds(base, C)],
                                           lg_v.at[pl.ds(e * C, C)], in_sem)
                     for e in range(E)]
        for c in in_copies:
            c.start()
        for c in in_copies:
            c.wait()

        def body(g, carry):
            t0 = g * 16
            vs = [lg_v[pl.ds(e * C + t0, 16)] for e in range(E)]
            best = vs[0]
            bi = jnp.zeros((16,), jnp.int32)
            for e in range(1, E):
                gt = vs[e] > best
                best = jnp.where(gt, vs[e], best)
                bi = jnp.where(gt, jnp.full((16,), e, jnp.int32), bi)
            best2 = jnp.full((16,), -jnp.inf, jnp.float32)
            bi2 = jnp.zeros((16,), jnp.int32)
            for e in range(E):
                ev = jnp.full((16,), e, jnp.int32)
                gt = (vs[e] > best2) & (bi != ev)
                best2 = jnp.where(gt, vs[e], best2)
                bi2 = jnp.where(gt, ev, bi2)
            ex = jnp.exp(best2 - best)
            w1 = 1.0 / (1.0 + ex)
            w2 = 1.0 - w1
            i1_v[pl.ds(t0, 16)] = bi
            i2_v[pl.ds(t0, 16)] = bi2
            w1_v[pl.ds(t0, 16)] = w1
            w2_v[pl.ds(t0, 16)] = w2
            return carry

        lax.fori_loop(0, G, body, 0)

        out_copies = [
            pltpu.make_async_copy(i1_v, i1_hbm.at[pl.ds(base, C)], out_sem),
            pltpu.make_async_copy(i2_v, i2_hbm.at[pl.ds(base, C)], out_sem),
            pltpu.make_async_copy(w1_v, w1_hbm.at[pl.ds(base, C)], out_sem),
            pltpu.make_async_copy(w2_v, w2_hbm.at[pl.ds(base, C)], out_sem),
        ]
        for c in out_copies:
            c.start()
        for c in out_copies:
            c.wait()

    return sc_kernel(logits_t)


def kernel(input, prototypes, input_ids, attention_mask):
    b, s, d = input.shape
    n = b * s
    x = input.astype(prototypes.dtype).reshape(n, d)
    logits_t = _tc_scaled_logits(x, prototypes)
    i1, i2, w1, w2 = _sc_topk_call(logits_t, n)
    top_idx = jnp.stack([i1, i2], axis=-1).reshape(b, s, 2)
    top_w = jnp.stack([w1, w2], axis=-1).reshape(b, s, 2)
    return top_idx, top_w
